# Initial kernel scaffold; baseline (speedup 1.0000x reference)
import jax
import jax.numpy as jnp
from jax.experimental import pallas as pl


def kernel(nodes, nlist, edges, inv_degree, w):
    raise NotImplementedError("write your pallas kernel here")


# speedup vs baseline: 1.2363x; 1.2363x over previous
import functools

import jax
import jax.numpy as jnp
from jax import lax
from jax.experimental import pallas as pl
from jax.experimental.pallas import tpu as pltpu
from jax.experimental.pallas import tpu_sc as plsc

N = 10000
K = 32
F = 128
DE = 4

NC = 2
NS = 16
NW = NC * NS

L = 16

C = 8
RPW = 320
NP = NW * RPW
NCHUNK = RPW // C

IDX_ROWS = (C * K) // 128


def _sc_stage(nodes, nlist_r, edges_p):
  mesh = plsc.VectorSubcoreMesh(core_axis_name="c", subcore_axis_name="s")

  @functools.partial(
      pl.kernel,
      mesh=mesh,
      out_type=jax.ShapeDtypeStruct((NP, DE * F), jnp.float32),
      scratch_types=[
          pltpu.VMEM((IDX_ROWS, 128), jnp.int32),
          pltpu.VMEM((C * K, F), jnp.float32),
          pltpu.VMEM((C, K * DE), jnp.float32),
          pltpu.VMEM((C, DE * F), jnp.float32),
          pltpu.SemaphoreType.DMA,
      ],
  )
  def sc_kernel(nodes_hbm, nlist_hbm, edges_hbm, u_hbm,
                idx_v, rows_v, e_v, u_v, sem):
    wid = lax.axis_index("s") * NC + lax.axis_index("c")
    row0 = wid * RPW

    def chunk_body(c, carry):
      base = row0 + c * C
      for p in range(IDX_ROWS):
        pltpu.sync_copy(nlist_hbm.at[pl.ds(base * K + p * 128, 128)],
                        idx_v.at[p])
      pltpu.sync_copy(edges_hbm.at[pl.ds(base, C)], e_v)
      cps = [
          pltpu.async_copy(nodes_hbm.at[idx_v.at[p]],
                           rows_v.at[pl.ds(p * 128, 128)], sem)
          for p in range(IDX_ROWS)
      ]
      for cp in cps:
        cp.wait()

      def row_body(r, carry2):
        g0 = r * K
        accs = [[jnp.zeros((L,), jnp.float32) for _ in range(F // L)]
                for _ in range(DE)]
        e_chunks = [e_v[r, pl.ds(q * L, L)] for q in range(K * DE // L)]
        for j in range(K):
          chunks = [rows_v[g0 + j, pl.ds(cs * L, L)] for cs in range(F // L)]
          for n in range(DE):
            t = j * DE + n
            eb = jnp.full((L,), e_chunks[t // L][t % L])
            for cs in range(F // L):
              accs[n][cs] = accs[n][cs] + eb * chunks[cs]
        for n in range(DE):
          for cs in range(F // L):
            u_v[r, pl.ds(n * F + cs * L, L)] = accs[n][cs]
        return carry2

      lax.fori_loop(0, C, row_body, 0, unroll=False)
      pltpu.sync_copy(u_v, u_hbm.at[pl.ds(base, C)])
      return carry

    lax.fori_loop(0, NCHUNK, chunk_body, 0, unroll=False)

  return sc_kernel(nodes, nlist_r, edges_p)


RBLK = 512


def _tc_body(u_ref, wmat_ref, deg_ref, o_ref):
  o_ref[...] = jnp.dot(u_ref[...], wmat_ref[...],
                       preferred_element_type=jnp.float32) * deg_ref[...]


def _tc_stage(u, wmat, deg_p):
  return pl.pallas_call(
      _tc_body,
      out_shape=jax.ShapeDtypeStruct((NP, F), jnp.float32),
      grid=(NP // RBLK,),
      in_specs=[
          pl.BlockSpec((RBLK, DE * F), lambda i: (i, 0)),
          pl.BlockSpec((DE * F, F), lambda i: (0, 0)),
          pl.BlockSpec((RBLK, 1), lambda i: (i, 0)),
      ],
      out_specs=pl.BlockSpec((RBLK, F), lambda i: (i, 0)),
  )(u, wmat, deg_p)


def kernel(nodes, nlist, edges, inv_degree, w):
  nlist = nlist.astype(jnp.int32)
  nlist_p = jnp.pad(nlist, ((0, NP - N), (0, 0)))
  nlist_r = nlist_p.reshape(NP * K)
  edges_p = jnp.pad(edges.reshape(N, K * DE), ((0, NP - N), (0, 0)))

  u = _sc_stage(nodes, nlist_r, edges_p)

  wmat = jnp.transpose(w, (2, 0, 1)).reshape(DE * F, F)
  deg_p = jnp.pad(inv_degree, (0, NP - N)).reshape(NP, 1)
  out = _tc_stage(u, wmat, deg_p)
  return out[:N]


# speedup vs baseline: 1.6547x; 1.3384x over previous
import functools

import jax
import jax.numpy as jnp
from jax import lax
from jax.experimental import pallas as pl
from jax.experimental.pallas import tpu as pltpu
from jax.experimental.pallas import tpu_sc as plsc

N = 10000
K = 32
F = 128
DE = 4

NC = 2
NS = 16
NW = NC * NS

L = 16

C = 8
RPW = 320
NP = NW * RPW
NCHUNK = RPW // C
NPP = NP + 2 * C

IDX_ROWS = (C * K) // 128


def _sc_stage(nodes, nlist_r, edges_p):
  mesh = plsc.VectorSubcoreMesh(core_axis_name="c", subcore_axis_name="s")

  @functools.partial(
      pl.kernel,
      mesh=mesh,
      out_type=jax.ShapeDtypeStruct((NP, DE * F), jnp.float32),
      scratch_types=[
          pltpu.VMEM((4, IDX_ROWS, 128), jnp.int32),
          pltpu.VMEM((2, C * K, F), jnp.float32),
          pltpu.VMEM((4, C, K * DE), jnp.float32),
          pltpu.VMEM((C, DE * F), jnp.float32),
          pltpu.SemaphoreType.DMA,
          pltpu.SemaphoreType.DMA,
          pltpu.SemaphoreType.DMA,
      ],
  )
  def sc_kernel(nodes_hbm, nlist_hbm, edges_hbm, u_hbm,
                idx_v, rows_v, e_v, u_v, ssem, gsem0, gsem1):
    wid = lax.axis_index("s") * NC + lax.axis_index("c")
    row0 = wid * RPW
    chunk0 = wid * NCHUNK

    gsems = (gsem0, gsem1)

    def stage_start(c, slot):
      pltpu.async_copy(nlist_hbm.at[chunk0 + c], idx_v.at[slot], ssem)
      pltpu.async_copy(edges_hbm.at[pl.ds(row0 + c * C, C)], e_v.at[slot],
                       ssem)

    def stage_wait(slot):
      pltpu.make_async_copy(nlist_hbm.at[0], idx_v.at[slot], ssem).wait()
      pltpu.make_async_copy(edges_hbm.at[pl.ds(0, C)], e_v.at[slot],
                            ssem).wait()

    def gather_start(slot):
      for p in range(IDX_ROWS):
        pltpu.async_copy(nodes_hbm.at[idx_v.at[slot, p]],
                         rows_v.at[slot % 2, pl.ds(p * 128, 128)],
                         gsems[slot % 2])

    def gather_wait(slot):
      for p in range(IDX_ROWS):
        pltpu.make_async_copy(nodes_hbm.at[idx_v.at[slot, p]],
                              rows_v.at[slot % 2, pl.ds(p * 128, 128)],
                              gsems[slot % 2]).wait()

    stage_start(0, 0)
    stage_wait(0)
    stage_start(1, 1)
    gather_start(0)

    def cc_body(cc, carry):
      for b in range(4):
        c = cc * 4 + b
        stage_wait((b + 1) % 4)
        gather_start((b + 1) % 4)
        stage_start(c + 2, (b + 2) % 4)
        gather_wait(b % 4)

        def row_body(r, carry2):
          g0 = r * K
          accs = [[jnp.zeros((L,), jnp.float32) for _ in range(F // L)]
                  for _ in range(DE)]
          e_chunks = [e_v[b, r, pl.ds(q * L, L)] for q in range(K * DE // L)]
          for j in range(K):
            chunks = [rows_v[b % 2, g0 + j, pl.ds(cs * L, L)]
                      for cs in range(F // L)]
            for n in range(DE):
              t = j * DE + n
              eb = jnp.full((L,), e_chunks[t // L][t % L])
              for cs in range(F // L):
                accs[n][cs] = accs[n][cs] + eb * chunks[cs]
          for n in range(DE):
            for cs in range(F // L):
              u_v[r, pl.ds(n * F + cs * L, L)] = accs[n][cs]
          return carry2

        lax.fori_loop(0, C, row_body, 0, unroll=False)
        pltpu.sync_copy(u_v, u_hbm.at[pl.ds(row0 + c * C, C)])
      return carry

    lax.fori_loop(0, NCHUNK // 4, cc_body, 0, unroll=False)

    stage_wait((NCHUNK + 1) % 4)
    gather_wait(NCHUNK % 4)

  return sc_kernel(nodes, nlist_r, edges_p)


RBLK = 512


def _tc_body(u_ref, wmat_ref, deg_ref, o_ref):
  o_ref[...] = jnp.dot(u_ref[...], wmat_ref[...],
                       preferred_element_type=jnp.float32) * deg_ref[...]


def _tc_stage(u, wmat, deg_p):
  return pl.pallas_call(
      _tc_body,
      out_shape=jax.ShapeDtypeStruct((NP, F), jnp.float32),
      grid=(NP // RBLK,),
      in_specs=[
          pl.BlockSpec((RBLK, DE * F), lambda i: (i, 0)),
          pl.BlockSpec((DE * F, F), lambda i: (0, 0)),
          pl.BlockSpec((RBLK, 1), lambda i: (i, 0)),
      ],
      out_specs=pl.BlockSpec((RBLK, F), lambda i: (i, 0)),
  )(u, wmat, deg_p)


def kernel(nodes, nlist, edges, inv_degree, w):
  nlist = nlist.astype(jnp.int32)
  nlist_p = jnp.pad(nlist, ((0, NPP - N), (0, 0)))
  nlist_r = nlist_p.reshape(NPP // C, IDX_ROWS, 128)
  edges_p = jnp.pad(edges.reshape(N, K * DE), ((0, NPP - N), (0, 0)))

  u = _sc_stage(nodes, nlist_r, edges_p)

  wmat = jnp.transpose(w, (2, 0, 1)).reshape(DE * F, F)
  deg_p = jnp.pad(inv_degree, (0, NP - N)).reshape(NP, 1)
  out = _tc_stage(u, wmat, deg_p)
  return out[:N]


# speedup vs baseline: 3.1915x; 1.9288x over previous
import functools

import jax
import jax.numpy as jnp
from jax import lax
from jax.experimental import pallas as pl
from jax.experimental.pallas import tpu as pltpu
from jax.experimental.pallas import tpu_sc as plsc

N = 10000
K = 32
F = 128
DE = 4

NC = 2
NS = 16
NW = NC * NS

L = 16

C = 4
RPW = 320
NP = NW * RPW
NCHUNK = RPW // C
NPP = NP + 2 * C
NNP = 10240

IDX_ROWS = (C * K) // 128


def _sc_stage(nodes, nlist_r, edges_p):
  mesh = plsc.VectorSubcoreMesh(core_axis_name="c", subcore_axis_name="s")

  @functools.partial(
      pl.kernel,
      mesh=mesh,
      out_type=jax.ShapeDtypeStruct((NP, DE * F), jnp.float32),
      scratch_types=[
          pltpu.VMEM((4, IDX_ROWS, 128), jnp.int32),
          pltpu.VMEM((2, C * K, F), jnp.float32),
          pltpu.VMEM((4, C, K * DE), jnp.float32),
          pltpu.VMEM((C, DE * F), jnp.float32),
          pltpu.VMEM_SHARED((NNP, F), jnp.float32),
          pltpu.SemaphoreType.DMA,
          pltpu.SemaphoreType.DMA,
          pltpu.SemaphoreType.DMA,
      ],
  )
  def sc_kernel(nodes_hbm, nlist_hbm, edges_hbm, u_hbm,
                idx_v, rows_v, e_v, u_v, nodes_sp, ssem, gsem0, gsem1):
    sid = lax.axis_index("s")
    wid = sid * NC + lax.axis_index("c")
    row0 = wid * RPW
    chunk0 = wid * NCHUNK

    nspt = NNP // NS
    stage0 = pl.multiple_of(sid * nspt, 8)
    pltpu.sync_copy(nodes_hbm.at[pl.ds(stage0, nspt)],
                    nodes_sp.at[pl.ds(stage0, nspt)])
    plsc.subcore_barrier()

    gsems = (gsem0, gsem1)

    def stage_start(c, slot):
      pltpu.async_copy(nlist_hbm.at[chunk0 + c], idx_v.at[slot], ssem)
      pltpu.async_copy(edges_hbm.at[pl.ds(row0 + c * C, C)], e_v.at[slot],
                       ssem)

    def stage_wait(slot):
      pltpu.make_async_copy(nlist_hbm.at[0], idx_v.at[slot], ssem).wait()
      pltpu.make_async_copy(edges_hbm.at[pl.ds(0, C)], e_v.at[slot],
                            ssem).wait()

    def gather_start(slot):
      for p in range(IDX_ROWS):
        pltpu.async_copy(nodes_sp.at[idx_v.at[slot, p]],
                         rows_v.at[slot % 2, pl.ds(p * 128, 128)],
                         gsems[slot % 2])

    def gather_wait(slot):
      for p in range(IDX_ROWS):
        pltpu.make_async_copy(nodes_sp.at[idx_v.at[slot, p]],
                              rows_v.at[slot % 2, pl.ds(p * 128, 128)],
                              gsems[slot % 2]).wait()

    stage_start(0, 0)
    stage_wait(0)
    stage_start(1, 1)
    gather_start(0)

    def cc_body(cc, carry):
      for b in range(4):
        c = cc * 4 + b
        stage_wait((b + 1) % 4)
        gather_start((b + 1) % 4)
        stage_start(c + 2, (b + 2) % 4)
        gather_wait(b % 4)

        def row_body(r, carry2):
          g0 = r * K
          accs = [[jnp.zeros((L,), jnp.float32) for _ in range(F // L)]
                  for _ in range(DE)]
          e_chunks = [e_v[b, r, pl.ds(q * L, L)] for q in range(K * DE // L)]
          for j in range(K):
            chunks = [rows_v[b % 2, g0 + j, pl.ds(cs * L, L)]
                      for cs in range(F // L)]
            for n in range(DE):
              t = j * DE + n
              eb = jnp.full((L,), e_chunks[t // L][t % L])
              for cs in range(F // L):
                accs[n][cs] = accs[n][cs] + eb * chunks[cs]
          for n in range(DE):
            for cs in range(F // L):
              u_v[r, pl.ds(n * F + cs * L, L)] = accs[n][cs]
          return carry2

        lax.fori_loop(0, C, row_body, 0, unroll=False)
        pltpu.sync_copy(u_v, u_hbm.at[pl.ds(row0 + c * C, C)])
      return carry

    lax.fori_loop(0, NCHUNK // 4, cc_body, 0, unroll=False)

    stage_wait((NCHUNK + 1) % 4)
    gather_wait(NCHUNK % 4)

  return sc_kernel(nodes, nlist_r, edges_p)


RBLK = 512


def _tc_body(u_ref, wmat_ref, deg_ref, o_ref):
  o_ref[...] = jnp.dot(u_ref[...], wmat_ref[...],
                       preferred_element_type=jnp.float32) * deg_ref[...]


def _tc_stage(u, wmat, deg_p):
  return pl.pallas_call(
      _tc_body,
      out_shape=jax.ShapeDtypeStruct((NP, F), jnp.float32),
      grid=(NP // RBLK,),
      in_specs=[
          pl.BlockSpec((RBLK, DE * F), lambda i: (i, 0)),
          pl.BlockSpec((DE * F, F), lambda i: (0, 0)),
          pl.BlockSpec((RBLK, 1), lambda i: (i, 0)),
      ],
      out_specs=pl.BlockSpec((RBLK, F), lambda i: (i, 0)),
  )(u, wmat, deg_p)


def kernel(nodes, nlist, edges, inv_degree, w):
  nodes = jnp.pad(nodes, ((0, NNP - N), (0, 0)))
  nlist = nlist.astype(jnp.int32)
  nlist_p = jnp.pad(nlist, ((0, NPP - N), (0, 0)))
  nlist_r = nlist_p.reshape(NPP // C, IDX_ROWS, 128)
  edges_p = jnp.pad(edges.reshape(N, K * DE), ((0, NPP - N), (0, 0)))

  u = _sc_stage(nodes, nlist_r, edges_p)

  wmat = jnp.transpose(w, (2, 0, 1)).reshape(DE * F, F)
  deg_p = jnp.pad(inv_degree, (0, NP - N)).reshape(NP, 1)
  out = _tc_stage(u, wmat, deg_p)
  return out[:N]


# speedup vs baseline: 3.4042x; 1.0666x over previous
import functools

import jax
import jax.numpy as jnp
from jax import lax
from jax.experimental import pallas as pl
from jax.experimental.pallas import tpu as pltpu
from jax.experimental.pallas import tpu_sc as plsc

N = 10000
K = 32
F = 128
DE = 4

NC = 2
NS = 16
NW = NC * NS

L = 16

C = 4
RPW = 320
NCHUNK = RPW // C
NCHUNK_ALL = (N * K) // (C * K)
NNP = 10240
NU = N + 2 * C


def _sc_stage(nodes, nlist_r, edges_r):
  mesh = plsc.VectorSubcoreMesh(core_axis_name="c", subcore_axis_name="s")

  @functools.partial(
      pl.kernel,
      mesh=mesh,
      out_type=jax.ShapeDtypeStruct((NU, DE * F), jnp.float32),
      scratch_types=[
          pltpu.VMEM((4, 1, C * K), jnp.int32),
          pltpu.VMEM((2, C * K, F), jnp.float32),
          pltpu.VMEM((4, C, K * DE), jnp.float32),
          pltpu.VMEM((2, C, DE * F), jnp.float32),
          pltpu.VMEM_SHARED((NNP, F), jnp.float32),
          pltpu.SemaphoreType.DMA,
          pltpu.SemaphoreType.DMA,
          pltpu.SemaphoreType.DMA,
          pltpu.SemaphoreType.DMA,
          pltpu.SemaphoreType.DMA,
      ],
  )
  def sc_kernel(nodes_hbm, nlist_hbm, edges_hbm, u_hbm,
                idx_v, rows_v, e_v, u_v, nodes_sp,
                ssem, gsem0, gsem1, wsem0, wsem1):
    sid = lax.axis_index("s")
    wid = sid * NC + lax.axis_index("c")
    chunk0 = wid * NCHUNK

    nspt = NNP // NS
    stage0 = pl.multiple_of(sid * nspt, 8)
    pltpu.sync_copy(nodes_hbm.at[pl.ds(stage0, nspt)],
                    nodes_sp.at[pl.ds(stage0, nspt)])
    plsc.subcore_barrier()

    gsems = (gsem0, gsem1)
    wsems = (wsem0, wsem1)

    def q_of(c):
      return jnp.minimum(chunk0 + c, NCHUNK_ALL - 1)

    def stage_start(c, slot):
      q = q_of(c)
      pltpu.async_copy(nlist_hbm.at[q], idx_v.at[slot], ssem)
      pltpu.async_copy(edges_hbm.at[pl.ds(q * C, C)], e_v.at[slot], ssem)

    def stage_wait(slot):
      pltpu.make_async_copy(nlist_hbm.at[0], idx_v.at[slot], ssem).wait()
      pltpu.make_async_copy(edges_hbm.at[pl.ds(0, C)], e_v.at[slot],
                            ssem).wait()

    def gather_start(slot):
      pltpu.async_copy(nodes_sp.at[idx_v.at[slot, 0]], rows_v.at[slot % 2],
                       gsems[slot % 2])

    def gather_wait(slot):
      pltpu.make_async_copy(nodes_sp.at[idx_v.at[slot, 0]],
                            rows_v.at[slot % 2], gsems[slot % 2]).wait()

    def wb_start(c, par):
      pltpu.async_copy(u_v.at[par], u_hbm.at[pl.ds(q_of(c) * C, C)],
                       wsems[par])

    def wb_wait(par):
      pltpu.make_async_copy(u_v.at[par], u_hbm.at[pl.ds(0, C)],
                            wsems[par]).wait()

    stage_start(0, 0)
    stage_wait(0)
    stage_start(1, 1)
    gather_start(0)
    pltpu.async_copy(u_v.at[0], u_hbm.at[pl.ds(N, C)], wsem0)
    pltpu.async_copy(u_v.at[1], u_hbm.at[pl.ds(N + C, C)], wsem1)

    def cc_body(cc, carry):
      for b in range(4):
        c = cc * 4 + b
        stage_wait((b + 1) % 4)
        gather_start((b + 1) % 4)
        stage_start(c + 2, (b + 2) % 4)
        wb_wait(b % 2)
        gather_wait(b % 4)

        def row_body(r, carry2):
          g0 = r * K
          accs = [[None for _ in range(F // L)] for _ in range(DE)]
          e_chunks = [e_v[b, r, pl.ds(q * L, L)]
                      for q in range(K * DE // L)]
          for j in range(K):
            chunks = [rows_v[b % 2, g0 + j, pl.ds(cs * L, L)]
                      for cs in range(F // L)]
            for n in range(DE):
              t = j * DE + n
              eb = jnp.full((L,), e_chunks[t // L][t % L])
              for cs in range(F // L):
                if j == 0:
                  accs[n][cs] = eb * chunks[cs]
                else:
                  accs[n][cs] = accs[n][cs] + eb * chunks[cs]
          for n in range(DE):
            for cs in range(F // L):
              u_v[b % 2, r, pl.ds(n * F + cs * L, L)] = accs[n][cs]
          return carry2

        lax.fori_loop(0, C, row_body, 0, unroll=False)
        wb_start(c, b % 2)
      return carry

    lax.fori_loop(0, NCHUNK // 4, cc_body, 0, unroll=False)

    stage_wait((NCHUNK + 1) % 4)
    gather_wait(NCHUNK % 4)
    wb_wait(0)
    wb_wait(1)

  return sc_kernel(nodes, nlist_r, edges_r)


RBLK = 1000


def _tc_body(u_ref, wmat_ref, deg_ref, o_ref):
  o_ref[...] = jnp.dot(u_ref[...], wmat_ref[...],
                       preferred_element_type=jnp.float32) * deg_ref[...]


def _tc_stage(u, wmat, deg):
  return pl.pallas_call(
      _tc_body,
      out_shape=jax.ShapeDtypeStruct((N, F), jnp.float32),
      grid=(N // RBLK,),
      in_specs=[
          pl.BlockSpec((RBLK, DE * F), lambda i: (i, 0)),
          pl.BlockSpec((DE * F, F), lambda i: (0, 0)),
          pl.BlockSpec((RBLK, 1), lambda i: (i, 0)),
      ],
      out_specs=pl.BlockSpec((RBLK, F), lambda i: (i, 0)),
  )(u, wmat, deg)


def kernel(nodes, nlist, edges, inv_degree, w):
  nodes = jnp.pad(nodes, ((0, NNP - N), (0, 0)))
  nlist_r = nlist.astype(jnp.int32).reshape(NCHUNK_ALL, 1, C * K)
  edges_r = edges.reshape(N, K * DE)

  u = _sc_stage(nodes, nlist_r, edges_r)

  wmat = jnp.transpose(w, (2, 0, 1)).reshape(DE * F, F)
  deg = inv_degree.reshape(N, 1)
  out = _tc_stage(u, wmat, deg)
  return out


# speedup vs baseline: 3.5418x; 1.0404x over previous
import functools

import jax
import jax.numpy as jnp
from jax import lax
from jax.experimental import pallas as pl
from jax.experimental.pallas import tpu as pltpu
from jax.experimental.pallas import tpu_sc as plsc

N = 10000
K = 32
F = 128
DE = 4

NC = 2
NS = 16
NW = NC * NS

L = 16

C = 4
RPW = 224
RSC = NW * RPW
NCHUNK = RPW // C
NCHUNK_ALL = RSC // C
NNP = 10240
NU = RSC + 2 * C

G = 80
B0 = 7120 // G
NTC = N - B0 * G


def _sc_stage(nodes, nlist_r, edges_r):
  mesh = plsc.VectorSubcoreMesh(core_axis_name="c", subcore_axis_name="s")

  @functools.partial(
      pl.kernel,
      mesh=mesh,
      out_type=jax.ShapeDtypeStruct((NU, DE * F), jnp.float32),
      scratch_types=[
          pltpu.VMEM((4, 1, C * K), jnp.int32),
          pltpu.VMEM((2, C * K, F), jnp.float32),
          pltpu.VMEM((4, C, K * DE), jnp.float32),
          pltpu.VMEM((2, C, DE * F), jnp.float32),
          pltpu.VMEM_SHARED((NNP, F), jnp.float32),
          pltpu.SemaphoreType.DMA,
          pltpu.SemaphoreType.DMA,
          pltpu.SemaphoreType.DMA,
          pltpu.SemaphoreType.DMA,
          pltpu.SemaphoreType.DMA,
      ],
  )
  def sc_kernel(nodes_hbm, nlist_hbm, edges_hbm, u_hbm,
                idx_v, rows_v, e_v, u_v, nodes_sp,
                ssem, gsem0, gsem1, wsem0, wsem1):
    sid = lax.axis_index("s")
    wid = sid * NC + lax.axis_index("c")
    chunk0 = wid * NCHUNK

    nspt = NNP // NS
    stage0 = pl.multiple_of(sid * nspt, 8)
    pltpu.sync_copy(nodes_hbm.at[pl.ds(stage0, nspt)],
                    nodes_sp.at[pl.ds(stage0, nspt)])
    plsc.subcore_barrier()

    gsems = (gsem0, gsem1)
    wsems = (wsem0, wsem1)

    def q_of(c):
      return jnp.minimum(chunk0 + c, NCHUNK_ALL - 1)

    def stage_start(c, slot):
      q = q_of(c)
      pltpu.async_copy(nlist_hbm.at[q], idx_v.at[slot], ssem)
      pltpu.async_copy(edges_hbm.at[pl.ds(q * C, C)], e_v.at[slot], ssem)

    def stage_wait(slot):
      pltpu.make_async_copy(nlist_hbm.at[0], idx_v.at[slot], ssem).wait()
      pltpu.make_async_copy(edges_hbm.at[pl.ds(0, C)], e_v.at[slot],
                            ssem).wait()

    def gather_start(slot):
      pltpu.async_copy(nodes_sp.at[idx_v.at[slot, 0]], rows_v.at[slot % 2],
                       gsems[slot % 2])

    def gather_wait(slot):
      pltpu.make_async_copy(nodes_sp.at[idx_v.at[slot, 0]],
                            rows_v.at[slot % 2], gsems[slot % 2]).wait()

    def wb_start(c, par):
      pltpu.async_copy(u_v.at[par], u_hbm.at[pl.ds(q_of(c) * C, C)],
                       wsems[par])

    def wb_wait(par):
      pltpu.make_async_copy(u_v.at[par], u_hbm.at[pl.ds(0, C)],
                            wsems[par]).wait()

    stage_start(0, 0)
    stage_wait(0)
    stage_start(1, 1)
    gather_start(0)
    pltpu.async_copy(u_v.at[0], u_hbm.at[pl.ds(RSC, C)], wsem0)
    pltpu.async_copy(u_v.at[1], u_hbm.at[pl.ds(RSC + C, C)], wsem1)

    def cc_body(cc, carry):
      for b in range(4):
        c = cc * 4 + b
        stage_wait((b + 1) % 4)
        gather_start((b + 1) % 4)
        stage_start(c + 2, (b + 2) % 4)
        wb_wait(b % 2)
        gather_wait(b % 4)

        def row_body(r, carry2):
          g0 = r * K
          accs = [[None for _ in range(F // L)] for _ in range(DE)]
          e_chunks = [e_v[b, r, pl.ds(q * L, L)]
                      for q in range(K * DE // L)]
          for j in range(K):
            chunks = [rows_v[b % 2, g0 + j, pl.ds(cs * L, L)]
                      for cs in range(F // L)]
            for n in range(DE):
              t = j * DE + n
              eb = jnp.full((L,), e_chunks[t // L][t % L])
              for cs in range(F // L):
                if j == 0:
                  accs[n][cs] = eb * chunks[cs]
                else:
                  accs[n][cs] = accs[n][cs] + eb * chunks[cs]
          for n in range(DE):
            for cs in range(F // L):
              u_v[b % 2, r, pl.ds(n * F + cs * L, L)] = accs[n][cs]
          return carry2

        lax.fori_loop(0, C, row_body, 0, unroll=False)
        wb_start(c, b % 2)
      return carry

    lax.fori_loop(0, NCHUNK // 4, cc_body, 0, unroll=False)

    stage_wait((NCHUNK + 1) % 4)
    gather_wait(NCHUNK % 4)
    wb_wait(0)
    wb_wait(1)

  return sc_kernel(nodes, nlist_r, edges_r)


MROWS = B0 * G
RBLK = MROWS // 10


def _tc_body(u_ref, wmat_ref, deg_ref, o_ref):
  o_ref[...] = jnp.dot(u_ref[...], wmat_ref[...],
                       preferred_element_type=jnp.float32) * deg_ref[...]


def _tc_stage(u, wmat, deg):
  return pl.pallas_call(
      _tc_body,
      out_shape=jax.ShapeDtypeStruct((MROWS, F), jnp.float32),
      grid=(MROWS // RBLK,),
      in_specs=[
          pl.BlockSpec((RBLK, DE * F), lambda i: (i, 0)),
          pl.BlockSpec((DE * F, F), lambda i: (0, 0)),
          pl.BlockSpec((RBLK, 1), lambda i: (i, 0)),
      ],
      out_specs=pl.BlockSpec((RBLK, F), lambda i: (i, 0)),
  )(u, wmat, deg)


def _tcp_body(nl_ref, nodes_ref, e2_ref, wmat_ref, deg_ref, o_ref,
              rows_s, tmp_s):
  def grp_body(g, carry):
    em_g = e2_ref[pl.ds(g * 8 * DE, 8 * DE), :]
    ts = []
    for s in range(8):
      r = g * 8 + s
      for j in range(K):
        idx = nl_ref[r, j]
        rows_s[pl.ds(j, 1), :] = nodes_ref[pl.ds(idx, 1), :]
      em = lax.slice(em_g, (s * DE, 0), ((s + 1) * DE, K))
      ts.append(jnp.dot(em, rows_s[...],
                        preferred_element_type=jnp.float32))
    for n in range(DE):
      blk = jnp.concatenate(
          [lax.slice(t, (n, 0), (n + 1, F)) for t in ts], axis=0)
      tmp_s[pl.ds(n * G + g * 8, 8), :] = blk
    return carry

  lax.fori_loop(0, G // 8, grp_body, 0, unroll=False)
  acc = jnp.dot(tmp_s[pl.ds(0, G), :], wmat_ref[pl.ds(0, F), :],
                preferred_element_type=jnp.float32)
  for n in range(1, DE):
    acc = acc + jnp.dot(tmp_s[pl.ds(n * G, G), :],
                        wmat_ref[pl.ds(n * F, F), :],
                        preferred_element_type=jnp.float32)
  o_ref[...] = acc * deg_ref[...]


def _tc_partial(nlist, nodes, e2, wmat, deg):
  return pl.pallas_call(
      _tcp_body,
      out_shape=jax.ShapeDtypeStruct((NTC, F), jnp.float32),
      grid=(NTC // G,),
      in_specs=[
          pl.BlockSpec((G, K), lambda i: (i + B0, 0),
                       memory_space=pltpu.SMEM),
          pl.BlockSpec(memory_space=pltpu.VMEM),
          pl.BlockSpec((G * DE, K), lambda i: (i + B0, 0)),
          pl.BlockSpec((DE * F, F), lambda i: (0, 0)),
          pl.BlockSpec((G, 1), lambda i: (i + B0, 0)),
      ],
      out_specs=pl.BlockSpec((G, F), lambda i: (i, 0)),
      scratch_shapes=[
          pltpu.VMEM((K, F), jnp.float32),
          pltpu.VMEM((DE * G, F), jnp.float32),
      ],
  )(nlist, nodes, e2, wmat, deg)


def kernel(nodes, nlist, edges, inv_degree, w):
  nodes_p = jnp.pad(nodes, ((0, NNP - N), (0, 0)))
  nlist = nlist.astype(jnp.int32)
  nlist_r = nlist.reshape((N * K) // (C * K), 1, C * K)
  edges_r = edges.reshape(N, K * DE)

  u = _sc_stage(nodes_p, nlist_r, edges_r)

  wmat = jnp.transpose(w, (2, 0, 1)).reshape(DE * F, F)
  deg = inv_degree.reshape(N, 1)
  e2 = jnp.transpose(edges, (0, 2, 1)).reshape(N * DE, K)
  out2 = _tc_partial(nlist, nodes, e2, wmat, deg)
  out1 = _tc_stage(u, wmat, deg)
  return jnp.concatenate([out1, out2], axis=0)


# speedup vs baseline: 3.5422x; 1.0001x over previous
import functools

import jax
import jax.numpy as jnp
from jax import lax
from jax.experimental import pallas as pl
from jax.experimental.pallas import tpu as pltpu
from jax.experimental.pallas import tpu_sc as plsc

N = 10000
K = 32
F = 128
DE = 4

NC = 2
NS = 16
NW = NC * NS

L = 16

C = 4
RPW = 224
RSC = NW * RPW
NCHUNK = RPW // C
NCHUNK_ALL = RSC // C
NNP = 10240
NU = RSC + 2 * C

G = 80
B0 = 7120 // G
NTC = N - B0 * G


def _sc_stage(nodes, nlist_r, edges_r):
  mesh = plsc.VectorSubcoreMesh(core_axis_name="c", subcore_axis_name="s")

  @functools.partial(
      pl.kernel,
      mesh=mesh,
      out_type=jax.ShapeDtypeStruct((NU, DE * F), jnp.float32),
      scratch_types=[
          pltpu.VMEM((4, 1, C * K), jnp.int32),
          pltpu.VMEM((2, C * K, F), jnp.float32),
          pltpu.VMEM((4, C, K * DE), jnp.float32),
          pltpu.VMEM((2, C, DE * F), jnp.float32),
          pltpu.VMEM_SHARED((NNP, F), jnp.float32),
          pltpu.SemaphoreType.DMA,
          pltpu.SemaphoreType.DMA,
          pltpu.SemaphoreType.DMA,
          pltpu.SemaphoreType.DMA,
          pltpu.SemaphoreType.DMA,
      ],
  )
  def sc_kernel(nodes_hbm, nlist_hbm, edges_hbm, u_hbm,
                idx_v, rows_v, e_v, u_v, nodes_sp,
                ssem, gsem0, gsem1, wsem0, wsem1):
    sid = lax.axis_index("s")
    wid = sid * NC + lax.axis_index("c")
    chunk0 = wid * NCHUNK

    nspt = NNP // NS
    stage0 = pl.multiple_of(sid * nspt, 8)
    pltpu.sync_copy(nodes_hbm.at[pl.ds(stage0, nspt)],
                    nodes_sp.at[pl.ds(stage0, nspt)])
    plsc.subcore_barrier()

    gsems = (gsem0, gsem1)
    wsems = (wsem0, wsem1)

    def q_of(c):
      return jnp.minimum(chunk0 + c, NCHUNK_ALL - 1)

    def stage_start(c, slot):
      q = q_of(c)
      pltpu.async_copy(nlist_hbm.at[q], idx_v.at[slot], ssem)
      pltpu.async_copy(edges_hbm.at[pl.ds(q * C, C)], e_v.at[slot], ssem)

    def stage_wait(slot):
      pltpu.make_async_copy(nlist_hbm.at[0], idx_v.at[slot], ssem).wait()
      pltpu.make_async_copy(edges_hbm.at[pl.ds(0, C)], e_v.at[slot],
                            ssem).wait()

    def gather_start(slot):
      pltpu.async_copy(nodes_sp.at[idx_v.at[slot, 0]], rows_v.at[slot % 2],
                       gsems[slot % 2])

    def gather_wait(slot):
      pltpu.make_async_copy(nodes_sp.at[idx_v.at[slot, 0]],
                            rows_v.at[slot % 2], gsems[slot % 2]).wait()

    def wb_start(c, par):
      pltpu.async_copy(u_v.at[par], u_hbm.at[pl.ds(q_of(c) * C, C)],
                       wsems[par])

    def wb_wait(par):
      pltpu.make_async_copy(u_v.at[par], u_hbm.at[pl.ds(0, C)],
                            wsems[par]).wait()

    stage_start(0, 0)
    stage_wait(0)
    stage_start(1, 1)
    gather_start(0)
    pltpu.async_copy(u_v.at[0], u_hbm.at[pl.ds(RSC, C)], wsem0)
    pltpu.async_copy(u_v.at[1], u_hbm.at[pl.ds(RSC + C, C)], wsem1)

    def cc_body(cc, carry):
      for b in range(4):
        c = cc * 4 + b
        stage_wait((b + 1) % 4)
        gather_start((b + 1) % 4)
        stage_start(c + 2, (b + 2) % 4)
        wb_wait(b % 2)
        gather_wait(b % 4)

        def row_body(r, carry2):
          g0 = r * K
          accs = [[None for _ in range(F // L)] for _ in range(DE)]
          e_chunks = [e_v[b, r, pl.ds(q * L, L)]
                      for q in range(K * DE // L)]
          for j in range(K):
            chunks = [rows_v[b % 2, g0 + j, pl.ds(cs * L, L)]
                      for cs in range(F // L)]
            for n in range(DE):
              t = j * DE + n
              eb = jnp.full((L,), e_chunks[t // L][t % L])
              for cs in range(F // L):
                if j == 0:
                  accs[n][cs] = eb * chunks[cs]
                else:
                  accs[n][cs] = accs[n][cs] + eb * chunks[cs]
          for n in range(DE):
            for cs in range(F // L):
              u_v[b % 2, r, pl.ds(n * F + cs * L, L)] = accs[n][cs]
          return carry2

        lax.fori_loop(0, C, row_body, 0, unroll=False)
        wb_start(c, b % 2)
      return carry

    lax.fori_loop(0, NCHUNK // 4, cc_body, 0, unroll=False)

    stage_wait((NCHUNK + 1) % 4)
    gather_wait(NCHUNK % 4)
    wb_wait(0)
    wb_wait(1)

  return sc_kernel(nodes, nlist_r, edges_r)


MROWS = B0 * G
RBLK = MROWS // 10


def _tc_body(u_ref, wmat_ref, deg_ref, o_ref):
  o_ref[...] = jnp.dot(u_ref[...], wmat_ref[...],
                       preferred_element_type=jnp.float32) * deg_ref[...]


def _tc_stage(u, wmat, deg):
  return pl.pallas_call(
      _tc_body,
      out_shape=jax.ShapeDtypeStruct((MROWS, F), jnp.float32),
      grid=(MROWS // RBLK,),
      in_specs=[
          pl.BlockSpec((RBLK, DE * F), lambda i: (i, 0)),
          pl.BlockSpec((DE * F, F), lambda i: (0, 0)),
          pl.BlockSpec((RBLK, 1), lambda i: (i, 0)),
      ],
      out_specs=pl.BlockSpec((RBLK, F), lambda i: (i, 0)),
  )(u, wmat, deg)


def _tcp_body(nl_ref, nodes_ref, e2_ref, wmat_ref, deg_ref, o_ref,
              rows_s, tmp_s):
  def grp_body(g, carry):
    em_g = e2_ref[pl.ds(g * 8 * DE, 8 * DE), :]
    ts = []
    for s in range(8):
      r = g * 8 + s
      h = (s % 2) * K
      for j in range(K):
        idx = nl_ref[r, j]
        rows_s[pl.ds(h + j, 1), :] = nodes_ref[pl.ds(idx, 1), :]
      em = lax.slice(em_g, (s * DE, 0), ((s + 1) * DE, K))
      ts.append(jnp.dot(em, rows_s[pl.ds(h, K), :],
                        preferred_element_type=jnp.float32))
    for n in range(DE):
      blk = jnp.concatenate(
          [lax.slice(t, (n, 0), (n + 1, F)) for t in ts], axis=0)
      tmp_s[pl.ds(n * G + g * 8, 8), :] = blk
    return carry

  lax.fori_loop(0, G // 8, grp_body, 0, unroll=False)
  acc = jnp.dot(tmp_s[pl.ds(0, G), :], wmat_ref[pl.ds(0, F), :],
                preferred_element_type=jnp.float32)
  for n in range(1, DE):
    acc = acc + jnp.dot(tmp_s[pl.ds(n * G, G), :],
                        wmat_ref[pl.ds(n * F, F), :],
                        preferred_element_type=jnp.float32)
  o_ref[...] = acc * deg_ref[...]


def _tc_partial(nlist, nodes, e2, wmat, deg):
  return pl.pallas_call(
      _tcp_body,
      out_shape=jax.ShapeDtypeStruct((NTC, F), jnp.float32),
      grid=(NTC // G,),
      in_specs=[
          pl.BlockSpec((G, K), lambda i: (i + B0, 0),
                       memory_space=pltpu.SMEM),
          pl.BlockSpec(memory_space=pltpu.VMEM),
          pl.BlockSpec((G * DE, K), lambda i: (i + B0, 0)),
          pl.BlockSpec((DE * F, F), lambda i: (0, 0)),
          pl.BlockSpec((G, 1), lambda i: (i + B0, 0)),
      ],
      out_specs=pl.BlockSpec((G, F), lambda i: (i, 0)),
      scratch_shapes=[
          pltpu.VMEM((2 * K, F), jnp.float32),
          pltpu.VMEM((DE * G, F), jnp.float32),
      ],
  )(nlist, nodes, e2, wmat, deg)


def kernel(nodes, nlist, edges, inv_degree, w):
  nodes_p = jnp.pad(nodes, ((0, NNP - N), (0, 0)))
  nlist = nlist.astype(jnp.int32)
  nlist_r = nlist.reshape((N * K) // (C * K), 1, C * K)
  edges_r = edges.reshape(N, K * DE)

  u = _sc_stage(nodes_p, nlist_r, edges_r)

  wmat = jnp.transpose(w, (2, 0, 1)).reshape(DE * F, F)
  deg = inv_degree.reshape(N, 1)
  e2 = jnp.transpose(edges, (0, 2, 1)).reshape(N * DE, K)
  out2 = _tc_partial(nlist, nodes, e2, wmat, deg)
  out1 = _tc_stage(u, wmat, deg)
  return jnp.concatenate([out1, out2], axis=0)


# speedup vs baseline: 3.9948x; 1.1278x over previous
import functools

import jax
import jax.numpy as jnp
from jax import lax
from jax.experimental import pallas as pl
from jax.experimental.pallas import tpu as pltpu
from jax.experimental.pallas import tpu_sc as plsc

N = 10000
K = 32
F = 128
DE = 4

NC = 2
NS = 16
NW = NC * NS

L = 16

C = 4
RPW = 240
RSC = NW * RPW
NCHUNK = RPW // C
NCHUNK_ALL = RSC // C
NU = RSC + 2 * C

G = 80
B0 = RSC // G
NTC = N - B0 * G


def _sc_stage(nodes, nlist_r, edges_r):
  mesh = plsc.VectorSubcoreMesh(core_axis_name="c", subcore_axis_name="s")

  @functools.partial(
      pl.kernel,
      mesh=mesh,
      out_type=jax.ShapeDtypeStruct((NU, DE * F), jnp.float32),
      scratch_types=[
          pltpu.VMEM((4, 1, C * K), jnp.int32),
          pltpu.VMEM((2, C * K, F), jnp.float32),
          pltpu.VMEM((4, C, K * DE), jnp.float32),
          pltpu.VMEM((2, C, DE * F), jnp.float32),
          pltpu.VMEM_SHARED((N, F), jnp.float32),
          pltpu.SemaphoreType.DMA,
          pltpu.SemaphoreType.DMA,
          pltpu.SemaphoreType.DMA,
          pltpu.SemaphoreType.DMA,
          pltpu.SemaphoreType.DMA,
      ],
  )
  def sc_kernel(nodes_hbm, nlist_hbm, edges_hbm, u_hbm,
                idx_v, rows_v, e_v, u_v, nodes_sp,
                ssem, gsem0, gsem1, wsem0, wsem1):
    sid = lax.axis_index("s")
    wid = sid * NC + lax.axis_index("c")
    chunk0 = wid * NCHUNK

    nspt = 640
    stage0 = pl.multiple_of(jnp.minimum(sid * nspt, N - nspt), 8)
    pltpu.sync_copy(nodes_hbm.at[pl.ds(stage0, nspt)],
                    nodes_sp.at[pl.ds(stage0, nspt)])
    plsc.subcore_barrier()

    gsems = (gsem0, gsem1)
    wsems = (wsem0, wsem1)

    def q_of(c):
      return jnp.minimum(chunk0 + c, NCHUNK_ALL - 1)

    def stage_start(c, slot):
      q = q_of(c)
      pltpu.async_copy(nlist_hbm.at[q], idx_v.at[slot], ssem)
      pltpu.async_copy(edges_hbm.at[pl.ds(q * C, C)], e_v.at[slot], ssem)

    def stage_wait(slot):
      pltpu.make_async_copy(nlist_hbm.at[0], idx_v.at[slot], ssem).wait()
      pltpu.make_async_copy(edges_hbm.at[pl.ds(0, C)], e_v.at[slot],
                            ssem).wait()

    def gather_start(slot):
      pltpu.async_copy(nodes_sp.at[idx_v.at[slot, 0]], rows_v.at[slot % 2],
                       gsems[slot % 2])

    def gather_wait(slot):
      pltpu.make_async_copy(nodes_sp.at[idx_v.at[slot, 0]],
                            rows_v.at[slot % 2], gsems[slot % 2]).wait()

    def wb_start(c, par):
      pltpu.async_copy(u_v.at[par], u_hbm.at[pl.ds(q_of(c) * C, C)],
                       wsems[par])

    def wb_wait(par):
      pltpu.make_async_copy(u_v.at[par], u_hbm.at[pl.ds(0, C)],
                            wsems[par]).wait()

    stage_start(0, 0)
    stage_wait(0)
    stage_start(1, 1)
    gather_start(0)
    pltpu.async_copy(u_v.at[0], u_hbm.at[pl.ds(RSC, C)], wsem0)
    pltpu.async_copy(u_v.at[1], u_hbm.at[pl.ds(RSC + C, C)], wsem1)

    def cc_body(cc, carry):
      for b in range(4):
        c = cc * 4 + b
        stage_wait((b + 1) % 4)
        gather_start((b + 1) % 4)
        stage_start(c + 2, (b + 2) % 4)
        wb_wait(b % 2)
        gather_wait(b % 4)

        def row_body(r, carry2):
          g0 = r * K
          accs = [[None for _ in range(F // L)] for _ in range(DE)]
          e_chunks = [e_v[b, r, pl.ds(q * L, L)]
                      for q in range(K * DE // L)]
          for j in range(K):
            chunks = [rows_v[b % 2, g0 + j, pl.ds(cs * L, L)]
                      for cs in range(F // L)]
            for n in range(DE):
              t = j * DE + n
              eb = jnp.full((L,), e_chunks[t // L][t % L])
              for cs in range(F // L):
                if j == 0:
                  accs[n][cs] = eb * chunks[cs]
                else:
                  accs[n][cs] = accs[n][cs] + eb * chunks[cs]
          for n in range(DE):
            for cs in range(F // L):
              u_v[b % 2, r, pl.ds(n * F + cs * L, L)] = accs[n][cs]
          return carry2

        lax.fori_loop(0, C, row_body, 0, unroll=False)
        wb_start(c, b % 2)
      return carry

    lax.fori_loop(0, NCHUNK // 4, cc_body, 0, unroll=False)

    stage_wait((NCHUNK + 1) % 4)
    gather_wait(NCHUNK % 4)
    wb_wait(0)
    wb_wait(1)

  return sc_kernel(nodes, nlist_r, edges_r)


MROWS = B0 * G
RBLK = MROWS // 10


def _tc_body(u_ref, wmat_ref, deg_ref, o_ref):
  o_ref[...] = jnp.dot(u_ref[...], wmat_ref[...],
                       preferred_element_type=jnp.float32) * deg_ref[...]


def _tc_stage(u, wmat, deg):
  return pl.pallas_call(
      _tc_body,
      out_shape=jax.ShapeDtypeStruct((MROWS, F), jnp.float32),
      grid=(MROWS // RBLK,),
      in_specs=[
          pl.BlockSpec((RBLK, DE * F), lambda i: (i, 0)),
          pl.BlockSpec((DE * F, F), lambda i: (0, 0)),
          pl.BlockSpec((RBLK, 1), lambda i: (i, 0)),
      ],
      out_specs=pl.BlockSpec((RBLK, F), lambda i: (i, 0)),
  )(u, wmat, deg)


def _tcp_body(nl_ref, nodes_ref, e2_ref, wmat_ref, deg_ref, o_ref,
              rows_s, tmp_s):
  def grp_body(g, carry):
    em_g = e2_ref[pl.ds(g * 8 * DE, 8 * DE), :]
    ts = []
    for s in range(8):
      r = g * 8 + s
      h = (s % 2) * K
      for j in range(K):
        idx = nl_ref[r, j]
        rows_s[pl.ds(h + j, 1), :] = nodes_ref[pl.ds(idx, 1), :]
      em = lax.slice(em_g, (s * DE, 0), ((s + 1) * DE, K))
      ts.append(jnp.dot(em, rows_s[pl.ds(h, K), :],
                        preferred_element_type=jnp.float32))
    for n in range(DE):
      blk = jnp.concatenate(
          [lax.slice(t, (n, 0), (n + 1, F)) for t in ts], axis=0)
      tmp_s[pl.ds(n * G + g * 8, 8), :] = blk
    return carry

  lax.fori_loop(0, G // 8, grp_body, 0, unroll=False)
  acc = jnp.dot(tmp_s[pl.ds(0, G), :], wmat_ref[pl.ds(0, F), :],
                preferred_element_type=jnp.float32)
  for n in range(1, DE):
    acc = acc + jnp.dot(tmp_s[pl.ds(n * G, G), :],
                        wmat_ref[pl.ds(n * F, F), :],
                        preferred_element_type=jnp.float32)
  o_ref[...] = acc * deg_ref[...]


def _tc_partial(nlist, nodes, e2, wmat, deg):
  return pl.pallas_call(
      _tcp_body,
      out_shape=jax.ShapeDtypeStruct((NTC, F), jnp.float32),
      grid=(NTC // G,),
      in_specs=[
          pl.BlockSpec((G, K), lambda i: (i + B0, 0),
                       memory_space=pltpu.SMEM),
          pl.BlockSpec(memory_space=pltpu.VMEM),
          pl.BlockSpec((G * DE, K), lambda i: (i + B0, 0)),
          pl.BlockSpec((DE * F, F), lambda i: (0, 0)),
          pl.BlockSpec((G, 1), lambda i: (i + B0, 0)),
      ],
      out_specs=pl.BlockSpec((G, F), lambda i: (i, 0)),
      scratch_shapes=[
          pltpu.VMEM((2 * K, F), jnp.float32),
          pltpu.VMEM((DE * G, F), jnp.float32),
      ],
  )(nlist, nodes, e2, wmat, deg)


def kernel(nodes, nlist, edges, inv_degree, w):
  nlist = nlist.astype(jnp.int32)
  nlist_r = nlist.reshape((N * K) // (C * K), 1, C * K)
  edges_r = edges.reshape(N, K * DE)

  u = _sc_stage(nodes, nlist_r, edges_r)

  wmat = jnp.transpose(w, (2, 0, 1)).reshape(DE * F, F)
  deg = inv_degree.reshape(N, 1)
  e2 = jnp.transpose(edges, (0, 2, 1)).reshape(N * DE, K)
  out2 = _tc_partial(nlist, nodes, e2, wmat, deg)
  out1 = _tc_stage(u, wmat, deg)
  return jnp.concatenate([out1, out2], axis=0)
